# Initial kernel scaffold; baseline (speedup 1.0000x reference)
import functools

import jax
import jax.numpy as jnp
from jax import lax
from jax.experimental import pallas as pl
from jax.experimental.pallas import tpu as pltpu
from jax.experimental.pallas import tpu_sc as plsc

_LANES = 16
_CHUNK_ROWS = 2


def _build(B, H, D, NC, NS):
    NW = NC * NS
    BW = B // NW
    NCH = BW // _CHUNK_ROWS
    CW = _CHUNK_ROWS * H

    mesh = plsc.VectorSubcoreMesh(core_axis_name="c", subcore_axis_name="s")

    @functools.partial(
        pl.kernel,
        mesh=mesh,
        out_type=jax.ShapeDtypeStruct((B, H), jnp.float32),
        scratch_types=[
            pltpu.VMEM((BW,), jnp.int32),
            pltpu.VMEM((NCH, CW), jnp.int32),
            pltpu.VMEM((BW, D), jnp.float32),
            pltpu.VMEM((CW, D), jnp.float32),
            pltpu.VMEM((BW, H), jnp.float32),
            pltpu.SemaphoreType.DMA,
        ],
    )
    def k(uidx_hbm, purch_hbm, cv_hbm, u_hbm, out_hbm,
          uidx_v, purch_v, urows_v, cbuf, out_v, sem):
        wid = lax.axis_index("s") * NC + lax.axis_index("c")
        base = wid * BW
        pltpu.sync_copy(uidx_hbm.at[pl.ds(base, BW)], uidx_v)
        pltpu.sync_copy(purch_hbm.at[pl.ds(wid * NCH, NCH)], purch_v)
        pltpu.async_copy(u_hbm.at[uidx_v], urows_v, sem).wait()

        def chunk_body(j, carry):
            pltpu.async_copy(cv_hbm.at[purch_v.at[j]], cbuf, sem).wait()
            for r in range(_CHUNK_ROWS):
                b = j * _CHUNK_ROWS + r
                uvec = [urows_v[b, pl.ds(kk * _LANES, _LANES)]
                        for kk in range(D // _LANES)]
                for l in range(H):
                    acc = cbuf[r * H + l, pl.ds(0, _LANES)] * uvec[0]
                    for kk in range(1, D // _LANES):
                        acc = acc + cbuf[r * H + l, pl.ds(kk * _LANES, _LANES)] * uvec[kk]
                    out_v[b, l] = jnp.sum(acc)
            return carry

        lax.fori_loop(0, NCH, chunk_body, 0)
        pltpu.sync_copy(out_v, out_hbm.at[pl.ds(base, BW)])

    return k


def kernel(user_idx, user_purchase, character_vector, u):
    B, H = user_purchase.shape
    D = u.shape[1]
    info = plsc.get_sparse_core_info()
    purch2 = user_purchase.reshape(B // _CHUNK_ROWS, _CHUNK_ROWS * H)
    k = _build(B, H, D, info.num_cores, info.num_subcores)
    return k(user_idx, purch2, character_vector, u)


# speedup vs baseline: 4.9463x; 4.9463x over previous
import functools

import jax
import jax.numpy as jnp
from jax import lax
from jax.experimental import pallas as pl
from jax.experimental.pallas import tpu as pltpu
from jax.experimental.pallas import tpu_sc as plsc

_LANES = 16
_CHUNK_ROWS = 2


def _build(B, H, D, NC, NS):
    NW = NC * NS
    BW = B // NW
    NCH = BW // _CHUNK_ROWS
    CW = _CHUNK_ROWS * H

    mesh = plsc.VectorSubcoreMesh(core_axis_name="c", subcore_axis_name="s")

    @functools.partial(
        pl.kernel,
        mesh=mesh,
        compiler_params=pltpu.CompilerParams(
            needs_layout_passes=False, use_tc_tiling_on_sc=False),
        out_type=jax.ShapeDtypeStruct((B, H), jnp.float32),
        scratch_types=[
            pltpu.VMEM((BW,), jnp.int32),
            pltpu.VMEM((NCH, CW), jnp.int32),
            pltpu.VMEM((BW, D), jnp.float32),
            pltpu.VMEM((CW, D), jnp.float32),
            pltpu.VMEM((BW, H), jnp.float32),
            pltpu.SemaphoreType.DMA,
        ],
    )
    def k(uidx_hbm, purch_hbm, cv_hbm, u_hbm, out_hbm,
          uidx_v, purch_v, urows_v, cbuf, out_v, sem):
        wid = lax.axis_index("s") * NC + lax.axis_index("c")
        base = wid * BW
        pltpu.sync_copy(uidx_hbm.at[pl.ds(base, BW)], uidx_v)
        pltpu.sync_copy(purch_hbm.at[pl.ds(wid * NCH, NCH)], purch_v)
        pltpu.async_copy(u_hbm.at[uidx_v], urows_v, sem).wait()

        group_offs = list(range(0, H - _LANES + 1, _LANES))
        if group_offs[-1] != H - _LANES:
            group_offs.append(H - _LANES)
        iota = jnp.arange(_LANES, dtype=jnp.int32)

        def chunk_body(j, carry):
            pltpu.async_copy(cv_hbm.at[purch_v.at[j]], cbuf, sem).wait()
            for r in range(_CHUNK_ROWS):
                b = j * _CHUNK_ROWS + r
                uvecs = [urows_v[b, pl.ds(kk * _LANES, _LANES)]
                         for kk in range(D // _LANES)]
                for l0 in group_offs:
                    outvec = jnp.zeros((_LANES,), jnp.float32)
                    for i in range(_LANES):
                        row = r * H + l0 + i
                        acc = cbuf[row, pl.ds(0, _LANES)] * uvecs[0]
                        for kk in range(1, D // _LANES):
                            acc = acc + cbuf[row, pl.ds(kk * _LANES, _LANES)] * uvecs[kk]
                        outvec = jnp.where(iota == i, jnp.sum(acc), outvec)
                    out_v[b, pl.ds(l0, _LANES)] = outvec
            return carry

        lax.fori_loop(0, NCH, chunk_body, 0)
        pltpu.sync_copy(out_v, out_hbm.at[pl.ds(base, BW)])

    return k


def kernel(user_idx, user_purchase, character_vector, u):
    B, H = user_purchase.shape
    D = u.shape[1]
    info = plsc.get_sparse_core_info()
    purch2 = user_purchase.reshape(B // _CHUNK_ROWS, _CHUNK_ROWS * H)
    k = _build(B, H, D, info.num_cores, info.num_subcores)
    return k(user_idx, purch2, character_vector, u)


# speedup vs baseline: 6.4772x; 1.3095x over previous
import functools

import jax
import jax.numpy as jnp
from jax import lax
from jax.experimental import pallas as pl
from jax.experimental.pallas import tpu as pltpu
from jax.experimental.pallas import tpu_sc as plsc

_LANES = 16
_STREAM_ROWS = 2
_CHUNK_STREAMS = 4
_CHUNK_ROWS = _STREAM_ROWS * _CHUNK_STREAMS


def _build(B, H, D, NC, NS):
    NW = NC * NS
    BW = B // NW
    NCH = BW // _CHUNK_ROWS
    SW = _STREAM_ROWS * H
    CE = _CHUNK_ROWS * H

    mesh = plsc.VectorSubcoreMesh(core_axis_name="c", subcore_axis_name="s")

    group_offs = list(range(0, H - _LANES + 1, _LANES))
    if group_offs[-1] != H - _LANES:
        group_offs.append(H - _LANES)

    @functools.partial(
        pl.kernel,
        mesh=mesh,
        compiler_params=pltpu.CompilerParams(
            needs_layout_passes=False, use_tc_tiling_on_sc=False),
        out_type=jax.ShapeDtypeStruct((B, H), jnp.float32),
        scratch_types=[
            pltpu.VMEM((BW,), jnp.int32),
            pltpu.VMEM((BW // _STREAM_ROWS, SW), jnp.int32),
            pltpu.VMEM((BW, D), jnp.float32),
            pltpu.VMEM((CE, D), jnp.float32),
            pltpu.VMEM((CE, D), jnp.float32),
            pltpu.VMEM((BW, H), jnp.float32),
            pltpu.SemaphoreType.DMA,
            pltpu.SemaphoreType.DMA,
            pltpu.SemaphoreType.DMA,
        ],
    )
    def k(uidx_hbm, purch_hbm, cv_hbm, u_hbm, out_hbm,
          uidx_v, purch_v, urows_v, buf_a, buf_b, out_v,
          sem_a, sem_b, sem_u):
        wid = lax.axis_index("s") * NC + lax.axis_index("c")
        base = wid * BW
        iota = jnp.arange(_LANES, dtype=jnp.int32)
        nstr = BW // _STREAM_ROWS
        pltpu.sync_copy(uidx_hbm.at[pl.ds(base, BW)], uidx_v)
        pltpu.sync_copy(purch_hbm.at[pl.ds(wid * nstr, nstr)], purch_v)

        def fire(j, buf, sem):
            for s in range(_CHUNK_STREAMS):
                pltpu.async_copy(
                    cv_hbm.at[purch_v.at[j * _CHUNK_STREAMS + s]],
                    buf.at[pl.ds(s * SW, SW)], sem)

        def drain(j, buf, sem):
            for s in range(_CHUNK_STREAMS):
                pltpu.make_async_copy(
                    cv_hbm.at[purch_v.at[j * _CHUNK_STREAMS + s]],
                    buf.at[pl.ds(s * SW, SW)], sem).wait()

        def compute(j, buf):
            def sub(s, carry):
                for r in range(_STREAM_ROWS):
                    row = j * _CHUNK_ROWS + s * _STREAM_ROWS + r
                    ce0 = s * SW + r * H
                    uvecs = [urows_v[row, pl.ds(kk * _LANES, _LANES)]
                             for kk in range(D // _LANES)]
                    for l0 in group_offs:
                        outvec = jnp.zeros((_LANES,), jnp.float32)
                        for i in range(_LANES):
                            e = ce0 + l0 + i
                            acc = buf[e, pl.ds(0, _LANES)] * uvecs[0]
                            for kk in range(1, D // _LANES):
                                acc = acc + buf[e, pl.ds(kk * _LANES, _LANES)] * uvecs[kk]
                            outvec = jnp.where(iota == i, jnp.sum(acc), outvec)
                        out_v[row, pl.ds(l0, _LANES)] = outvec
                return carry
            lax.fori_loop(0, _CHUNK_STREAMS, sub, 0)

        fire(0, buf_a, sem_a)
        pltpu.async_copy(u_hbm.at[uidx_v], urows_v, sem_u)
        pltpu.make_async_copy(u_hbm.at[uidx_v], urows_v, sem_u).wait()

        def body(jj, carry):
            j0 = 2 * jj
            j1 = 2 * jj + 1
            fire(j1, buf_b, sem_b)
            drain(j0, buf_a, sem_a)
            compute(j0, buf_a)

            @pl.when(j1 + 1 < NCH)
            def _():
                fire(j1 + 1, buf_a, sem_a)

            drain(j1, buf_b, sem_b)
            compute(j1, buf_b)
            return carry

        lax.fori_loop(0, NCH // 2, body, 0)
        pltpu.sync_copy(out_v, out_hbm.at[pl.ds(base, BW)])

    return k


def kernel(user_idx, user_purchase, character_vector, u):
    B, H = user_purchase.shape
    D = u.shape[1]
    info = plsc.get_sparse_core_info()
    purch2 = user_purchase.reshape(B // _STREAM_ROWS, _STREAM_ROWS * H)
    k = _build(B, H, D, info.num_cores, info.num_subcores)
    return k(user_idx, purch2, character_vector, u)
